# baseline (speedup 1.0000x reference)
import functools

import jax
import jax.numpy as jnp
from jax.experimental import pallas as pl
from jax.experimental.pallas import tpu as pltpu

_NE = 64
_EPS = 1e-6
_TILE = 4096


def _pass1_body(x_ref, w_ref, b_ref, v_ref, e_ref, r_ref, dacc_ref, *,
                num_tiles, capacity):
    i = pl.program_id(0)
    lt = jax.lax.dot_general(
        w_ref[...], x_ref[...], (((1,), (1,)), ((), ())),
        preferred_element_type=jnp.float32)
    lt = lt + b_ref[...]
    m = jnp.max(lt, axis=0, keepdims=True)
    s = jnp.sum(jnp.exp(lt - m), axis=0, keepdims=True)
    v = 1.0 / s
    iota = jax.lax.broadcasted_iota(jnp.int32, (_NE, _TILE), 0)
    e = jnp.min(jnp.where(lt == m, iota, _NE), axis=0, keepdims=True)
    v_ref[...] = v[0][:, None]
    e_ref[...] = e[0][:, None]
    contrib = jnp.where(iota == e, v, 0.0)

    @pl.when(i == 0)
    def _():
        dacc_ref[...] = jnp.zeros_like(dacc_ref)

    dacc_ref[...] += contrib

    @pl.when(i == num_tiles - 1)
    def _():
        denom = jnp.sum(dacc_ref[...], axis=1) + _EPS
        r_ref[0, :] = capacity / denom


def _pass2_body(v_ref, e_ref, r_ref, o_ref):
    recip = r_ref[...]
    v = v_ref[...]
    e = e_ref[...]
    iota = jax.lax.broadcasted_iota(jnp.int32, (_TILE, _NE), 1)
    o_ref[...] = jnp.where(iota == e, v * recip, 0.0)


def kernel(x, w_gate, b_gate):
    n, dim = x.shape
    ne = w_gate.shape[0]
    capacity = float(n)
    num_tiles = n // _TILE
    b2 = b_gate.reshape(ne, 1)

    vc, ec, recip = pl.pallas_call(
        functools.partial(_pass1_body, num_tiles=num_tiles,
                          capacity=capacity),
        grid=(num_tiles,),
        in_specs=[
            pl.BlockSpec((_TILE, dim), lambda i: (i, 0)),
            pl.BlockSpec((ne, dim), lambda i: (0, 0)),
            pl.BlockSpec((ne, 1), lambda i: (0, 0)),
        ],
        out_specs=[
            pl.BlockSpec((_TILE, 1), lambda i: (i, 0)),
            pl.BlockSpec((_TILE, 1), lambda i: (i, 0)),
            pl.BlockSpec((1, ne), lambda i: (0, 0)),
        ],
        out_shape=[
            jax.ShapeDtypeStruct((n, 1), jnp.float32),
            jax.ShapeDtypeStruct((n, 1), jnp.int32),
            jax.ShapeDtypeStruct((1, ne), jnp.float32),
        ],
        scratch_shapes=[pltpu.VMEM((_NE, _TILE), jnp.float32)],
        compiler_params=pltpu.CompilerParams(
            dimension_semantics=("arbitrary",)),
    )(x, w_gate, b2)

    out = pl.pallas_call(
        _pass2_body,
        grid=(num_tiles,),
        in_specs=[
            pl.BlockSpec((_TILE, 1), lambda i: (i, 0)),
            pl.BlockSpec((_TILE, 1), lambda i: (i, 0)),
            pl.BlockSpec((1, ne), lambda i: (0, 0)),
        ],
        out_specs=pl.BlockSpec((_TILE, ne), lambda i: (i, 0)),
        out_shape=jax.ShapeDtypeStruct((n, ne), jnp.float32),
        compiler_params=pltpu.CompilerParams(
            dimension_semantics=("arbitrary",)),
    )(vc, ec, recip)
    return out


# speedup vs baseline: 1.3835x; 1.3835x over previous
import functools

import jax
import jax.numpy as jnp
from jax.experimental import pallas as pl
from jax.experimental.pallas import tpu as pltpu

_NE = 64
_EPS = 1e-6
_TILE = 4096
_TILE2 = 8192


def _pass1_body(x_ref, w_ref, b_ref, v_ref, e_ref, r_ref, dacc_ref, *,
                num_tiles, capacity):
    i = pl.program_id(0)
    lt = jax.lax.dot_general(
        w_ref[...], x_ref[...], (((1,), (1,)), ((), ())),
        preferred_element_type=jnp.float32)
    lt = lt + b_ref[...]
    m = jnp.max(lt, axis=0, keepdims=True)
    s = jnp.sum(jnp.exp(lt - m), axis=0, keepdims=True)
    v = 1.0 / s
    iota = jax.lax.broadcasted_iota(jnp.int32, (_NE, _TILE), 0)
    e = jnp.min(jnp.where(lt == m, iota, _NE), axis=0, keepdims=True)
    v_ref[0, 0, :] = v[0]
    e_ref[0, 0, :] = e[0]
    contrib = jnp.where(iota == e, v, 0.0)

    @pl.when(i == 0)
    def _():
        dacc_ref[...] = jnp.zeros_like(dacc_ref)

    dacc_ref[...] += contrib

    @pl.when(i == num_tiles - 1)
    def _():
        denom = jnp.sum(dacc_ref[...], axis=1) + _EPS
        r_ref[0, :] = capacity / denom


def _pass2_body(v_ref, e_ref, r_ref, o_ref):
    recip = r_ref[...].reshape(_NE, 1)
    v = v_ref[0, 0, :][None, :]
    e = e_ref[0, 0, :][None, :]
    iota = jax.lax.broadcasted_iota(jnp.int32, (_NE, _TILE2), 0)
    out_t = jnp.where(iota == e, v * recip, 0.0)
    o_ref[...] = out_t.T


def kernel(x, w_gate, b_gate):
    n, dim = x.shape
    ne = w_gate.shape[0]
    capacity = float(n)
    num_tiles = n // _TILE
    b2 = b_gate.reshape(ne, 1)

    v3, e3, recip = pl.pallas_call(
        functools.partial(_pass1_body, num_tiles=num_tiles,
                          capacity=capacity),
        grid=(num_tiles,),
        in_specs=[
            pl.BlockSpec((_TILE, dim), lambda i: (i, 0)),
            pl.BlockSpec((ne, dim), lambda i: (0, 0)),
            pl.BlockSpec((ne, 1), lambda i: (0, 0)),
        ],
        out_specs=[
            pl.BlockSpec((1, 1, _TILE), lambda i: (i, 0, 0)),
            pl.BlockSpec((1, 1, _TILE), lambda i: (i, 0, 0)),
            pl.BlockSpec((1, ne), lambda i: (0, 0)),
        ],
        out_shape=[
            jax.ShapeDtypeStruct((num_tiles, 1, _TILE), jnp.float32),
            jax.ShapeDtypeStruct((num_tiles, 1, _TILE), jnp.int32),
            jax.ShapeDtypeStruct((1, ne), jnp.float32),
        ],
        scratch_shapes=[pltpu.VMEM((_NE, _TILE), jnp.float32)],
        compiler_params=pltpu.CompilerParams(
            dimension_semantics=("arbitrary",)),
    )(x, w_gate, b2)

    num_tiles2 = n // _TILE2
    out = pl.pallas_call(
        _pass2_body,
        grid=(num_tiles2,),
        in_specs=[
            pl.BlockSpec((1, 1, _TILE2), lambda i: (i, 0, 0)),
            pl.BlockSpec((1, 1, _TILE2), lambda i: (i, 0, 0)),
            pl.BlockSpec((1, ne), lambda i: (0, 0)),
        ],
        out_specs=pl.BlockSpec((_TILE2, ne), lambda i: (i, 0)),
        out_shape=jax.ShapeDtypeStruct((n, ne), jnp.float32),
        compiler_params=pltpu.CompilerParams(
            dimension_semantics=("arbitrary",)),
    )(v3.reshape(num_tiles2, 1, _TILE2), e3.reshape(num_tiles2, 1, _TILE2),
      recip)
    return out


# speedup vs baseline: 1.3985x; 1.0109x over previous
import functools

import jax
import jax.numpy as jnp
from jax.experimental import pallas as pl
from jax.experimental.pallas import tpu as pltpu

_NE = 64
_EPS = 1e-6
_TILE = 4096


def _body(x_ref, w_ref, b_ref, o_ref, v_s, e_s, dacc, r_s, *,
          num_tiles, capacity):
    p = pl.program_id(0)
    i = pl.program_id(1)

    @pl.when(p == 0)
    def _phase0():
        lt = jax.lax.dot_general(
            w_ref[...], x_ref[...], (((1,), (1,)), ((), ())),
            preferred_element_type=jnp.float32)
        lt = lt + b_ref[...]
        m = jnp.max(lt, axis=0, keepdims=True)
        s = jnp.sum(jnp.exp(lt - m), axis=0, keepdims=True)
        v = 1.0 / s
        iota = jax.lax.broadcasted_iota(jnp.int32, (_NE, _TILE), 0)
        e = jnp.min(jnp.where(lt == m, iota, _NE), axis=0, keepdims=True)
        v_s[i, 0, :] = v[0]
        e_s[i, 0, :] = e[0]
        contrib = jnp.where(iota == e, v, 0.0)

        @pl.when(i == 0)
        def _():
            dacc[...] = jnp.zeros_like(dacc)

        dacc[...] += contrib

        @pl.when(i == num_tiles - 1)
        def _():
            denom = jnp.sum(dacc[...], axis=1) + _EPS
            r_s[...] = (capacity / denom)[:, None]

    @pl.when(p == 1)
    def _phase1():
        recip = r_s[...]
        v = v_s[i]
        e = e_s[i]
        iota = jax.lax.broadcasted_iota(jnp.int32, (_NE, _TILE), 0)
        out_t = jnp.where(iota == e, v * recip, 0.0)
        o_ref[...] = out_t.T


def kernel(x, w_gate, b_gate):
    n, dim = x.shape
    ne = w_gate.shape[0]
    capacity = float(n)
    num_tiles = n // _TILE
    b2 = b_gate.reshape(ne, 1)
    last = num_tiles - 1

    out = pl.pallas_call(
        functools.partial(_body, num_tiles=num_tiles, capacity=capacity),
        grid=(2, num_tiles),
        in_specs=[
            pl.BlockSpec((_TILE, dim), lambda p, i: ((1 - p) * i + p * last,
                                                     0)),
            pl.BlockSpec((ne, dim), lambda p, i: (0, 0)),
            pl.BlockSpec((ne, 1), lambda p, i: (0, 0)),
        ],
        out_specs=pl.BlockSpec((_TILE, ne), lambda p, i: (p * i, 0)),
        out_shape=jax.ShapeDtypeStruct((n, ne), jnp.float32),
        scratch_shapes=[
            pltpu.VMEM((n // _TILE, 1, _TILE), jnp.float32),
            pltpu.VMEM((n // _TILE, 1, _TILE), jnp.int32),
            pltpu.VMEM((_NE, _TILE), jnp.float32),
            pltpu.VMEM((_NE, 1), jnp.float32),
        ],
        compiler_params=pltpu.CompilerParams(
            dimension_semantics=("arbitrary", "arbitrary")),
    )(x, w_gate, b2)
    return out
